# baseline (speedup 1.0000x reference)
import functools

import jax
import jax.numpy as jnp
from jax import lax
from jax.experimental import pallas as pl
from jax.experimental.pallas import tpu as pltpu
from jax.experimental.pallas import tpu_sc as plsc

_K = 512
_B = 64
_N = 8192
_L = 16
_NW = 32
_ROWS_PER_W = _B // _NW
_CHUNKS = _N // _L
_UNROLL = 8

_NB = 512
_LOWM = (1 << 22) - 1
_REG = 257

_GATHER_DNUMS = lax.GatherDimensionNumbers(
    offset_dims=(), collapsed_slice_dims=(0,), start_index_map=(0,))


def _rot(v, idx):
    return lax.gather(v, idx[:, None], dimension_numbers=_GATHER_DNUMS,
                      slice_sizes=(1,),
                      mode=lax.GatherScatterMode.PROMISE_IN_BOUNDS)


def _lane_sum(v):
    iota = lax.iota(jnp.int32, _L)
    for shift in (8, 4, 2, 1):
        v = v + _rot(v, (iota + shift) & (_L - 1))
    return v


def _lane_max(v):
    iota = lax.iota(jnp.int32, _L)
    for shift in (8, 4, 2, 1):
        v = jnp.maximum(v, _rot(v, (iota + shift) & (_L - 1)))
    return v


def _body(x_hbm, out_hbm, row_v, bits_v, hist_a, hist_b, ss_v,
          regs_a, regs_b, out_v):
    wid = lax.axis_index("s") * 2 + lax.axis_index("c")
    iota = lax.iota(jnp.int32, _L)
    ones = jnp.ones((_L,), jnp.int32)
    zeros = jnp.zeros((_L,), jnp.int32)
    kvec = jnp.full((_L,), _K, jnp.int32)

    for r in range(_ROWS_PER_W):
        row = wid * _ROWS_PER_W + r
        pltpu.sync_copy(x_hbm.at[row], row_v)

        def clear(i, c):
            for u in range(_UNROLL):
                off = (i * _UNROLL + u) * _L
                hist_a[pl.ds(off, _L)] = zeros
                hist_b[pl.ds(off, _L)] = zeros
            return c

        with jax.named_scope("ph_clear"):
            lax.fori_loop(0, (_NB + 1) // _UNROLL + 1, clear, jnp.int32(0))

        def prep(i, c):
            for u in range(_UNROLL):
                off = (i * _UNROLL + u) * _L
                b = lax.bitcast_convert_type(jnp.abs(row_v[pl.ds(off, _L)]),
                                             jnp.int32)
                bits_v[pl.ds(off, _L)] = b
                idx = ((b >> 22) << 4) + iota
                if u % 2 == 0:
                    plsc.addupdate_scatter(hist_a, [idx], ones)
                else:
                    plsc.addupdate_scatter(hist_b, [idx], ones)
            return c

        with jax.named_scope("ph_prep"):
            lax.fori_loop(0, _CHUNKS // _UNROLL, prep, jnp.int32(0))

        ss_v[pl.ds(_NB * _L, _L)] = zeros

        def scan(i, acc):
            base = (31 - i) * (_L * _L)
            for u in range(_L - 1, -1, -1):
                acc = (acc + hist_a[pl.ds(base + u * _L, _L)]
                       + hist_b[pl.ds(base + u * _L, _L)])
                ss_v[pl.ds(base + u * _L, _L)] = acc
            return acc

        with jax.named_scope("ph_scan"):
            lax.fori_loop(0, _NB // _L, scan, zeros)

        def bstep(i, p):
            cand = p + (jnp.int32(1) << (jnp.int32(8) - i))
            tot = _lane_sum(ss_v[pl.ds(cand * _L, _L)])
            return jnp.where(tot[0] >= _K, cand, p)

        with jax.named_scope("ph_bsearch"):
            p = lax.fori_loop(0, 9, bstep, jnp.int32(0))
        p_vec = jnp.broadcast_to(p, (_L,))
        c_above = _lane_sum(ss_v[pl.ds((p + 1) * _L, _L)])
        kp_vec = kvec - c_above

        def comp(i, carry):
            offs_a, offs_b = carry
            for u in range(_UNROLL):
                off = (i * _UNROLL + u) * _L
                b = bits_v[pl.ds(off, _L)]
                m = (b >> 22) == p_vec
                if u % 2 == 0:
                    plsc.store_scatter(regs_a, [offs_a], b & _LOWM, mask=m)
                    offs_a = offs_a + jnp.where(m, ones, zeros)
                else:
                    plsc.store_scatter(regs_b, [offs_b], b & _LOWM, mask=m)
                    offs_b = offs_b + jnp.where(m, ones, zeros)
            return offs_a, offs_b

        with jax.named_scope("ph_compact"):
            offs_a, offs_b = lax.fori_loop(0, _CHUNKS // _UNROLL, comp,
                                           (iota * _REG, iota * _REG))
        cnts_a = offs_a - iota * _REG
        cnts_b = offs_b - iota * _REG
        nch = (_lane_max(jnp.maximum(cnts_a, cnts_b))[0] + (_L - 1)) // _L

        def tclear(j, c):
            pos = j * _L + iota
            for l in range(_L):
                base = l * _REG + j * _L
                ca = jnp.broadcast_to(cnts_a[l], (_L,))
                olda = regs_a[pl.ds(base, _L)]
                regs_a[pl.ds(base, _L)] = jnp.where(pos < ca, olda, zeros)
                cb = jnp.broadcast_to(cnts_b[l], (_L,))
                oldb = regs_b[pl.ds(base, _L)]
                regs_b[pl.ds(base, _L)] = jnp.where(pos < cb, oldb, zeros)
            return c

        with jax.named_scope("ph_tclear"):
            lax.fori_loop(0, nch, tclear, jnp.int32(0))

        def bit_step(bi, t):
            cand = t | (ones << jnp.broadcast_to(jnp.int32(21) - bi, (_L,)))

            def cch(j, cnt):
                for l in range(_L):
                    base = l * _REG + j * _L
                    va = regs_a[pl.ds(base, _L)]
                    cnt = cnt + jnp.where(va >= cand, ones, zeros)
                    vb = regs_b[pl.ds(base, _L)]
                    cnt = cnt + jnp.where(vb >= cand, ones, zeros)
                return cnt

            cnt = lax.fori_loop(0, nch, cch, zeros)
            tot = _lane_sum(cnt)
            return jnp.where(tot >= kp_vec, cand, t)

        with jax.named_scope("ph_refine"):
            tlo = lax.fori_loop(0, 22, bit_step, zeros)
        thresh = (p_vec << 22) | tlo

        def mask_chunk(i, c):
            for u in range(_UNROLL):
                off = (i * _UNROLL + u) * _L
                v = row_v[pl.ds(off, _L)]
                keep = lax.bitcast_convert_type(jnp.abs(v), jnp.int32) >= thresh
                out_v[pl.ds(off, _L)] = jnp.where(keep, v, jnp.float32(0))
            return c

        with jax.named_scope("ph_mask"):
            lax.fori_loop(0, _CHUNKS // _UNROLL, mask_chunk, jnp.int32(0))
        pltpu.sync_copy(out_v, out_hbm.at[row])


@jax.jit
def kernel(x):
    mesh = plsc.VectorSubcoreMesh(core_axis_name="c", subcore_axis_name="s")
    fn = functools.partial(
        pl.kernel,
        mesh=mesh,
        compiler_params=pltpu.CompilerParams(needs_layout_passes=False),
        out_type=jax.ShapeDtypeStruct((_B, _N), jnp.float32),
        scratch_types=[
            pltpu.VMEM((_N,), jnp.float32),
            pltpu.VMEM((_N,), jnp.int32),
            pltpu.VMEM(((_NB + _UNROLL) * _L,), jnp.int32),
            pltpu.VMEM(((_NB + _UNROLL) * _L,), jnp.int32),
            pltpu.VMEM(((_NB + _UNROLL) * _L,), jnp.int32),
            pltpu.VMEM((_REG * _L,), jnp.int32),
            pltpu.VMEM((_REG * _L,), jnp.int32),
            pltpu.VMEM((_N,), jnp.float32),
        ],
    )(_body)
    return fn(x)


# speedup vs baseline: 1.0716x; 1.0716x over previous
import functools

import jax
import jax.numpy as jnp
from jax import lax
from jax.experimental import pallas as pl
from jax.experimental.pallas import tpu as pltpu
from jax.experimental.pallas import tpu_sc as plsc

_K = 512
_B = 64
_N = 8192
_L = 16
_L2 = 32
_NW = 32
_ROWS_PER_W = _B // _NW
_CHUNKS = _N // _L
_CHUNKS2 = _N // _L2
_UNROLL = 8

_GATHER_DNUMS = lax.GatherDimensionNumbers(
    offset_dims=(), collapsed_slice_dims=(0,), start_index_map=(0,))


def _rot(v, idx):
    return lax.gather(v, idx[:, None], dimension_numbers=_GATHER_DNUMS,
                      slice_sizes=(1,),
                      mode=lax.GatherScatterMode.PROMISE_IN_BOUNDS)


def _lane_sum(v):
    iota = lax.iota(jnp.int32, _L)
    for shift in (8, 4, 2, 1):
        v = v + _rot(v, (iota + shift) & (_L - 1))
    return v


def _ge_total(cnt32):
    s = _lane_sum((cnt32 & 0xFFFF) + (cnt32 >> 16))
    return _N - s


def _splat16(v32):
    return plsc.bitcast(v32 | (v32 << 16), jnp.int16)


def _body(x_hbm, out_hbm, row_v, hi_v, lo_v, out_v):
    wid = lax.axis_index("s") * 2 + lax.axis_index("c")
    ones = jnp.ones((_L,), jnp.int32)
    zeros = jnp.zeros((_L,), jnp.int32)
    kvec = jnp.full((_L,), _K, jnp.int32)

    for r in range(_ROWS_PER_W):
        row = wid * _ROWS_PER_W + r
        pltpu.sync_copy(x_hbm.at[row], row_v)

        def prep(i, c):
            for u in range(0, _UNROLL, 2):
                off = (i * _UNROLL + u) * _L
                b0 = lax.bitcast_convert_type(jnp.abs(row_v[pl.ds(off, _L)]),
                                              jnp.int32)
                b1 = lax.bitcast_convert_type(
                    jnp.abs(row_v[pl.ds(off + _L, _L)]), jnp.int32)
                hi_v[pl.ds(off, _L2)] = plsc.pack(
                    b0 >> 16, b1 >> 16, format=plsc.PackFormat.INTERLEAVED)
                lo_v[pl.ds(off, _L2)] = plsc.pack(
                    (b0 >> 1) & 0x7FFF, (b1 >> 1) & 0x7FFF,
                    format=plsc.PackFormat.INTERLEAVED)
            return c

        with jax.named_scope("ph_prep"):
            lax.fori_loop(0, _CHUNKS // _UNROLL, prep, jnp.int32(0))

        def make_search(arr_v):
            def step(bi, t):
                cand = t | (ones << jnp.broadcast_to(jnp.int32(14) - bi,
                                                     (_L,)))
                cand16 = _splat16(cand)

                def cch(j, cnt):
                    for u in range(_UNROLL):
                        off = (j * _UNROLL + u) * _L2
                        d = plsc.bitcast(arr_v[pl.ds(off, _L2)] - cand16,
                                         jnp.int32)
                        cnt = cnt + ((d >> 15) & 0x00010001)
                    return cnt

                cnt = lax.fori_loop(0, _CHUNKS2 // _UNROLL, cch, zeros)
                tot = _ge_total(cnt)
                return jnp.where(tot >= kvec, cand, t)
            return step

        with jax.named_scope("ph_hisearch"):
            t1 = lax.fori_loop(0, 15, make_search(hi_v), zeros)

        t16 = _splat16(t1)

        def fold(j, c):
            for u in range(_UNROLL):
                off = (j * _UNROLL + u) * _L2
                h = hi_v[pl.ds(off, _L2)]
                l32 = plsc.bitcast(lo_v[pl.ds(off, _L2)], jnp.int32)
                pl_ = (plsc.bitcast(h - t16, jnp.int32) >> 15) & 0x00010001
                pg = (plsc.bitcast(t16 - h, jnp.int32) >> 15) & 0x00010001
                ltm = (pl_ << 16) - pl_
                gtm = (pg << 16) - pg
                eqm = ~(ltm | gtm)
                res = (l32 & eqm) | (gtm & 0x7FFF7FFF) | ltm
                lo_v[pl.ds(off, _L2)] = plsc.bitcast(res, jnp.int16)
            return c

        with jax.named_scope("ph_fold"):
            lax.fori_loop(0, _CHUNKS2 // _UNROLL, fold, jnp.int32(0))

        with jax.named_scope("ph_losearch"):
            t2 = lax.fori_loop(0, 15, make_search(lo_v), zeros)

        cand0 = (t1 << 16) | (t2 << 1) | 1

        def last(i, cnt):
            for u in range(_UNROLL):
                off = (i * _UNROLL + u) * _L
                b = lax.bitcast_convert_type(jnp.abs(row_v[pl.ds(off, _L)]),
                                             jnp.int32)
                cnt = cnt + jnp.where(b >= cand0, ones, zeros)
            return cnt

        with jax.named_scope("ph_lastbit"):
            cnt = lax.fori_loop(0, _CHUNKS // _UNROLL, last, zeros)
        tot = _lane_sum(cnt)
        thresh = jnp.where(tot >= kvec, cand0, cand0 & ~1)

        def mask_chunk(i, c):
            for u in range(_UNROLL):
                off = (i * _UNROLL + u) * _L
                v = row_v[pl.ds(off, _L)]
                keep = lax.bitcast_convert_type(jnp.abs(v), jnp.int32) >= thresh
                out_v[pl.ds(off, _L)] = jnp.where(keep, v, jnp.float32(0))
            return c

        with jax.named_scope("ph_mask"):
            lax.fori_loop(0, _CHUNKS // _UNROLL, mask_chunk, jnp.int32(0))
        pltpu.sync_copy(out_v, out_hbm.at[row])


@jax.jit
def kernel(x):
    mesh = plsc.VectorSubcoreMesh(core_axis_name="c", subcore_axis_name="s")
    fn = functools.partial(
        pl.kernel,
        mesh=mesh,
        compiler_params=pltpu.CompilerParams(needs_layout_passes=False,
                                             use_tc_tiling_on_sc=False),
        out_type=jax.ShapeDtypeStruct((_B, _N), jnp.float32),
        scratch_types=[
            pltpu.VMEM((_N,), jnp.float32),
            pltpu.VMEM((_N,), jnp.int16),
            pltpu.VMEM((_N,), jnp.int16),
            pltpu.VMEM((_N,), jnp.float32),
        ],
    )(_body)
    return fn(x)


# speedup vs baseline: 1.1910x; 1.1114x over previous
import functools

import jax
import jax.numpy as jnp
from jax import lax
from jax.experimental import pallas as pl
from jax.experimental.pallas import tpu as pltpu
from jax.experimental.pallas import tpu_sc as plsc

_K = 512
_B = 64
_N = 8192
_L = 16
_L2 = 32
_NW = 32
_ROWS_PER_W = _B // _NW
_CHUNKS = _N // _L
_CHUNKS2 = _N // _L2
_UNROLL = 8

_GATHER_DNUMS = lax.GatherDimensionNumbers(
    offset_dims=(), collapsed_slice_dims=(0,), start_index_map=(0,))


def _rot(v, idx):
    return lax.gather(v, idx[:, None], dimension_numbers=_GATHER_DNUMS,
                      slice_sizes=(1,),
                      mode=lax.GatherScatterMode.PROMISE_IN_BOUNDS)


def _lane_sum(v):
    iota = lax.iota(jnp.int32, _L)
    for shift in (8, 4, 2, 1):
        v = v + _rot(v, (iota + shift) & (_L - 1))
    return v


def _ge_total(cnt32):
    s = _lane_sum((cnt32 & 0xFFFF) + (cnt32 >> 16))
    return _N - s


def _splat16(v32):
    return plsc.bitcast(v32 | (v32 << 16), jnp.int16)


def _body(x_hbm, out_hbm, row_a, row_b, hi_v, lo_v, lsem_a, lsem_b, ssem):
    wid = lax.axis_index("s") * 2 + lax.axis_index("c")
    ones = jnp.ones((_L,), jnp.int32)
    zeros = jnp.zeros((_L,), jnp.int32)
    kvec = jnp.full((_L,), _K, jnp.int32)

    row0 = wid * _ROWS_PER_W
    stores = []
    cp_a = pltpu.make_async_copy(x_hbm.at[row0], row_a, lsem_a)
    cp_b = pltpu.make_async_copy(x_hbm.at[row0 + 1], row_b, lsem_b)
    cp_a.start()
    cp_b.start()

    for r in range(_ROWS_PER_W):
        row_v = row_a if r == 0 else row_b
        (cp_a if r == 0 else cp_b).wait()

        def prep(i, c):
            for u in range(0, _UNROLL, 2):
                off = (i * _UNROLL + u) * _L
                b0 = lax.bitcast_convert_type(jnp.abs(row_v[pl.ds(off, _L)]),
                                              jnp.int32)
                b1 = lax.bitcast_convert_type(
                    jnp.abs(row_v[pl.ds(off + _L, _L)]), jnp.int32)
                o2 = off >> 1
                hi_v[pl.ds(o2, _L)] = (b0 >> 16) | (b1 & 0x7FFF0000)
                lo_v[pl.ds(o2, _L)] = ((b0 >> 1) & 0x7FFF) | ((b1 << 15)
                                                              & 0x7FFF0000)
            return c

        with jax.named_scope("ph_prep"):
            lax.fori_loop(0, _CHUNKS // _UNROLL, prep, jnp.int32(0))

        def make_search(arr_v):
            def step(bi, t):
                cand = t | (ones << jnp.broadcast_to(jnp.int32(14) - bi,
                                                     (_L,)))
                cand16 = _splat16(cand)

                def cch(j, cnt):
                    for u in range(_UNROLL):
                        off = (j * _UNROLL + u) * _L
                        v16 = plsc.bitcast(arr_v[pl.ds(off, _L)], jnp.int16)
                        d = plsc.bitcast(v16 - cand16, jnp.int32)
                        cnt = cnt + ((d >> 15) & 0x00010001)
                    return cnt

                cnt = lax.fori_loop(0, _CHUNKS2 // _UNROLL, cch, zeros)
                tot = _ge_total(cnt)
                return jnp.where(tot >= kvec, cand, t)
            return step

        with jax.named_scope("ph_hisearch"):
            t1 = lax.fori_loop(0, 15, make_search(hi_v), zeros)

        t16 = _splat16(t1)

        def fold(j, c):
            for u in range(_UNROLL):
                off = (j * _UNROLL + u) * _L
                h = plsc.bitcast(hi_v[pl.ds(off, _L)], jnp.int16)
                l32 = lo_v[pl.ds(off, _L)]
                pl_ = (plsc.bitcast(h - t16, jnp.int32) >> 15) & 0x00010001
                pg = (plsc.bitcast(t16 - h, jnp.int32) >> 15) & 0x00010001
                ltm = (pl_ << 16) - pl_
                gtm = (pg << 16) - pg
                eqm = ~(ltm | gtm)
                lo_v[pl.ds(off, _L)] = (l32 & eqm) | (gtm & 0x7FFF7FFF) | ltm
            return c

        with jax.named_scope("ph_fold"):
            lax.fori_loop(0, _CHUNKS2 // _UNROLL, fold, jnp.int32(0))

        with jax.named_scope("ph_losearch"):
            t2 = lax.fori_loop(0, 15, make_search(lo_v), zeros)

        cand0 = (t1 << 16) | (t2 << 1) | 1

        def last(i, cnt):
            for u in range(_UNROLL):
                off = (i * _UNROLL + u) * _L
                b = lax.bitcast_convert_type(jnp.abs(row_v[pl.ds(off, _L)]),
                                             jnp.int32)
                cnt = cnt + jnp.where(b >= cand0, ones, zeros)
            return cnt

        with jax.named_scope("ph_lastbit"):
            cnt = lax.fori_loop(0, _CHUNKS // _UNROLL, last, zeros)
        tot = _lane_sum(cnt)
        thresh = jnp.where(tot >= kvec, cand0, cand0 & ~1)

        def mask_chunk(i, c):
            for u in range(_UNROLL):
                off = (i * _UNROLL + u) * _L
                v = row_v[pl.ds(off, _L)]
                keep = lax.bitcast_convert_type(jnp.abs(v), jnp.int32) >= thresh
                row_v[pl.ds(off, _L)] = jnp.where(keep, v, jnp.float32(0))
            return c

        with jax.named_scope("ph_mask"):
            lax.fori_loop(0, _CHUNKS // _UNROLL, mask_chunk, jnp.int32(0))
        st = pltpu.make_async_copy(row_v, out_hbm.at[row0 + r], ssem)
        st.start()
        stores.append(st)

    for st in stores:
        st.wait()


@jax.jit
def kernel(x):
    mesh = plsc.VectorSubcoreMesh(core_axis_name="c", subcore_axis_name="s")
    fn = functools.partial(
        pl.kernel,
        mesh=mesh,
        compiler_params=pltpu.CompilerParams(needs_layout_passes=False),
        out_type=jax.ShapeDtypeStruct((_B, _N), jnp.float32),
        scratch_types=[
            pltpu.VMEM((_N,), jnp.float32),
            pltpu.VMEM((_N,), jnp.float32),
            pltpu.VMEM((_N // 2,), jnp.int32),
            pltpu.VMEM((_N // 2,), jnp.int32),
            pltpu.SemaphoreType.DMA,
            pltpu.SemaphoreType.DMA,
            pltpu.SemaphoreType.DMA,
        ],
    )(_body)
    return fn(x)


# speedup vs baseline: 1.2735x; 1.0693x over previous
import functools

import jax
import jax.numpy as jnp
from jax import lax
from jax.experimental import pallas as pl
from jax.experimental.pallas import tpu as pltpu
from jax.experimental.pallas import tpu_sc as plsc

_K = 512
_B = 64
_N = 8192
_L = 16
_L2 = 32
_NW = 32
_ROWS_PER_W = _B // _NW
_CHUNKS = _N // _L
_CHUNKS2 = _N // _L2
_UNROLL = 8

_GATHER_DNUMS = lax.GatherDimensionNumbers(
    offset_dims=(), collapsed_slice_dims=(0,), start_index_map=(0,))


def _rot(v, idx):
    return lax.gather(v, idx[:, None], dimension_numbers=_GATHER_DNUMS,
                      slice_sizes=(1,),
                      mode=lax.GatherScatterMode.PROMISE_IN_BOUNDS)


def _lane_sum(v):
    iota = lax.iota(jnp.int32, _L)
    for shift in (8, 4, 2, 1):
        v = v + _rot(v, (iota + shift) & (_L - 1))
    return v


def _ge_total(cnt32):
    s = _lane_sum((cnt32 & 0xFFFF) + (cnt32 >> 16))
    return _N - s


def _splat16(v32):
    return plsc.bitcast(v32 | (v32 << 16), jnp.int16)


def _body(x_hbm, out_hbm, row_a, row_b, hi_v, lo_v, lsem_a, lsem_b, ssem):
    wid = lax.axis_index("s") * 2 + lax.axis_index("c")
    ones = jnp.ones((_L,), jnp.int32)
    zeros = jnp.zeros((_L,), jnp.int32)
    kvec = jnp.full((_L,), _K, jnp.int32)

    row0 = wid * _ROWS_PER_W
    stores = []
    cp_a = pltpu.make_async_copy(x_hbm.at[row0], row_a, lsem_a)
    cp_b = pltpu.make_async_copy(x_hbm.at[row0 + 1], row_b, lsem_b)
    cp_a.start()
    cp_b.start()

    for r in range(_ROWS_PER_W):
        row_v = row_a if r == 0 else row_b
        (cp_a if r == 0 else cp_b).wait()

        with jax.named_scope("ph_prep"):
            @plsc.parallel_loop(0, _CHUNKS // 2, unroll=_UNROLL,
                                carry=jnp.int32(0))
            def _prep(i, c):
                off = i * _L2
                b0 = lax.bitcast_convert_type(jnp.abs(row_v[pl.ds(off, _L)]),
                                              jnp.int32)
                b1 = lax.bitcast_convert_type(
                    jnp.abs(row_v[pl.ds(off + _L, _L)]), jnp.int32)
                o2 = i * _L
                hi_v[pl.ds(o2, _L)] = (b0 >> 16) | (b1 & 0x7FFF0000)
                lo_v[pl.ds(o2, _L)] = ((b0 >> 1) & 0x7FFF) | ((b1 << 15)
                                                              & 0x7FFF0000)
                return c

        def make_search(arr_v):
            def step(bi, t):
                cand = t | (ones << jnp.broadcast_to(jnp.int32(14) - bi,
                                                     (_L,)))
                cand16 = _splat16(cand)

                @plsc.parallel_loop(0, _CHUNKS2, unroll=_UNROLL,
                                    carry=zeros)
                def cnt(j, cnt):
                    v16 = plsc.bitcast(arr_v[pl.ds(j * _L, _L)], jnp.int16)
                    d = plsc.bitcast(v16 - cand16, jnp.int32)
                    return cnt + ((d >> 15) & 0x00010001)
                tot = _ge_total(cnt)
                return jnp.where(tot >= kvec, cand, t)
            return step

        with jax.named_scope("ph_hisearch"):
            t1 = lax.fori_loop(0, 15, make_search(hi_v), zeros)

        t16 = _splat16(t1)

        with jax.named_scope("ph_fold"):
            @plsc.parallel_loop(0, _CHUNKS2, unroll=_UNROLL,
                                carry=jnp.int32(0))
            def _fold(j, c):
                off = j * _L
                h = plsc.bitcast(hi_v[pl.ds(off, _L)], jnp.int16)
                l32 = lo_v[pl.ds(off, _L)]
                pl_ = (plsc.bitcast(h - t16, jnp.int32) >> 15) & 0x00010001
                pg = (plsc.bitcast(t16 - h, jnp.int32) >> 15) & 0x00010001
                ltm = (pl_ << 16) - pl_
                gtm = (pg << 16) - pg
                eqm = ~(ltm | gtm)
                lo_v[pl.ds(off, _L)] = (l32 & eqm) | (gtm & 0x7FFF7FFF) | ltm
                return c

        with jax.named_scope("ph_losearch"):
            t2 = lax.fori_loop(0, 15, make_search(lo_v), zeros)

        cand0 = (t1 << 16) | (t2 << 1) | 1

        with jax.named_scope("ph_lastbit"):
            @plsc.parallel_loop(0, _CHUNKS, unroll=_UNROLL, carry=zeros)
            def cnt(i, cnt):
                b = lax.bitcast_convert_type(
                    jnp.abs(row_v[pl.ds(i * _L, _L)]), jnp.int32)
                return cnt + jnp.where(b >= cand0, ones, zeros)
        tot = _lane_sum(cnt)
        thresh = jnp.where(tot >= kvec, cand0, cand0 & ~1)

        with jax.named_scope("ph_mask"):
            @plsc.parallel_loop(0, _CHUNKS, unroll=_UNROLL,
                                carry=jnp.int32(0))
            def _mask(i, c):
                v = row_v[pl.ds(i * _L, _L)]
                keep = lax.bitcast_convert_type(jnp.abs(v), jnp.int32) >= thresh
                row_v[pl.ds(i * _L, _L)] = jnp.where(keep, v, jnp.float32(0))
                return c
        st = pltpu.make_async_copy(row_v, out_hbm.at[row0 + r], ssem)
        st.start()
        stores.append(st)

    for st in stores:
        st.wait()


@jax.jit
def kernel(x):
    mesh = plsc.VectorSubcoreMesh(core_axis_name="c", subcore_axis_name="s")
    fn = functools.partial(
        pl.kernel,
        mesh=mesh,
        compiler_params=pltpu.CompilerParams(needs_layout_passes=False),
        out_type=jax.ShapeDtypeStruct((_B, _N), jnp.float32),
        scratch_types=[
            pltpu.VMEM((_N,), jnp.float32),
            pltpu.VMEM((_N,), jnp.float32),
            pltpu.VMEM((_N // 2,), jnp.int32),
            pltpu.VMEM((_N // 2,), jnp.int32),
            pltpu.SemaphoreType.DMA,
            pltpu.SemaphoreType.DMA,
            pltpu.SemaphoreType.DMA,
        ],
    )(_body)
    return fn(x)
